# baseline (speedup 1.0000x reference)
import functools

import jax
import jax.numpy as jnp
from jax import lax
from jax.experimental import pallas as pl
from jax.experimental.pallas import tpu as pltpu
from jax.experimental.pallas import tpu_sc as plsc

DIM = 128
L = 200
LP = 224
LPH = LP // 2
EPS = 1e-12

NCORES = 2
NSUB = 16
NW = NCORES * NSUB
NCH = DIM // 16
IBLK = 32


def _sc_pool(W, idx_flat, rows):
    rows_per_w = rows // NW
    rows_per_sc = rows // NCORES
    mesh = plsc.VectorSubcoreMesh(core_axis_name="c", subcore_axis_name="s")

    @functools.partial(
        pl.kernel,
        out_type=jax.ShapeDtypeStruct((rows, DIM), jnp.float32),
        mesh=mesh,
        scratch_types=[
            pltpu.VMEM((IBLK * LP,), jnp.int32),
            pltpu.VMEM((LPH, DIM), jnp.float32),
            pltpu.VMEM((LPH, DIM), jnp.float32),
            pltpu.VMEM((LPH, DIM), jnp.float32),
            pltpu.VMEM((LPH, DIM), jnp.float32),
            pltpu.VMEM((LPH,), jnp.int32),
            pltpu.VMEM_SHARED((rows_per_sc, DIM), jnp.float32),
            pltpu.SemaphoreType.DMA,
            pltpu.SemaphoreType.DMA,
            pltpu.SemaphoreType.DMA,
            pltpu.SemaphoreType.DMA,
        ],
    )
    def pool_kernel(w_hbm, idx_hbm, out_hbm, idx_all, buf_a, buf_b, buf_c,
                    buf_d, didx, acc_sh, g0, g1, g2, g3):
        cid = lax.axis_index("c")
        sid = lax.axis_index("s")
        wid = cid * NSUB + sid
        base = wid * rows_per_w
        lbase = sid * rows_per_w

        def load_iblk(first_row):
            off = pl.multiple_of((base + first_row) * LP, 8)
            pltpu.sync_copy(idx_hbm.at[pl.ds(off, IBLK * LP)], idx_all)

        load_iblk(0)

        zero = jnp.zeros((16,), jnp.float32)

        @pl.loop(0, LPH)
        def _(l):
            for c in range(NCH):
                buf_a[l, pl.ds(c * 16, 16)] = zero

        pltpu.sync_copy(buf_a, acc_sh.at[pl.ds(lbase, LPH)])
        pltpu.sync_copy(buf_a, acc_sh.at[pl.ds(lbase + LPH, LPH)])
        pltpu.sync_copy(buf_a.at[pl.ds(0, rows_per_w - 2 * LPH)],
                        acc_sh.at[pl.ds(lbase + 2 * LPH,
                                        rows_per_w - 2 * LPH)])

        def fire(rr, half, buf, sem):
            off = pl.multiple_of(lax.rem(rr, IBLK) * LP + half * LPH, 8)
            return pltpu.async_copy(
                w_hbm.at[idx_all.at[pl.ds(off, LPH)]], buf, sem)

        def wait(buf, sem):
            pltpu.make_async_copy(w_hbm.at[idx_all.at[pl.ds(0, LPH)]],
                                  buf, sem).wait()

        def fill_didx(rr):
            slot = jnp.broadcast_to((lbase + rr).astype(jnp.int32), (16,))
            for k in range(LPH // 16):
                didx[pl.ds(k * 16, 16)] = slot

        fire(0, 0, buf_a, g0)
        fire(0, 1, buf_b, g1)
        fire(1, 0, buf_c, g2)
        fire(1, 1, buf_d, g3)

        @pl.loop(0, rows_per_w, step=2)
        def _(r0):
            wait(buf_a, g0)
            wait(buf_b, g1)
            fill_didx(r0)
            d0 = pltpu.async_copy(buf_a, acc_sh.at[didx], g0, add=True)
            d1 = pltpu.async_copy(buf_b, acc_sh.at[didx], g1, add=True)
            d0.wait()
            d1.wait()

            @pl.when(jnp.logical_and(lax.rem(r0 + 2, IBLK) == 0,
                                     r0 + 2 < rows_per_w))
            def _():
                load_iblk(r0 + 2)

            @pl.when(r0 + 2 < rows_per_w)
            def _():
                fire(r0 + 2, 0, buf_a, g0)
                fire(r0 + 2, 1, buf_b, g1)

            wait(buf_c, g2)
            wait(buf_d, g3)
            fill_didx(r0 + 1)
            d2 = pltpu.async_copy(buf_c, acc_sh.at[didx], g2, add=True)
            d3 = pltpu.async_copy(buf_d, acc_sh.at[didx], g3, add=True)
            d2.wait()
            d3.wait()

            @pl.when(r0 + 3 < rows_per_w)
            def _():
                fire(r0 + 3, 0, buf_c, g2)
                fire(r0 + 3, 1, buf_d, g3)

        pltpu.sync_copy(acc_sh.at[pl.ds(lbase, rows_per_w)],
                        out_hbm.at[pl.ds(base, rows_per_w)])

    return pool_kernel(W, idx_flat)


def _tc_norm(psum, idx, gamma, beta, rows):
    blk = 256

    def body(ps_ref, idx_ref, g_ref, b_ref, o_ref):
        s = ps_ref[...]
        cnt = jnp.sum((idx_ref[...] != 0).astype(jnp.float32), axis=1,
                      keepdims=True)
        p = s / cnt
        mu = jnp.mean(p, axis=1, keepdims=True)
        var = jnp.mean((p - mu) ** 2, axis=1, keepdims=True)
        o_ref[...] = (p - mu) * lax.rsqrt(var + EPS) * g_ref[...] + b_ref[...]

    return pl.pallas_call(
        body,
        grid=(rows // blk,),
        in_specs=[
            pl.BlockSpec((blk, DIM), lambda i: (i, 0)),
            pl.BlockSpec((blk, L), lambda i: (i, 0)),
            pl.BlockSpec((1, DIM), lambda i: (0, 0)),
            pl.BlockSpec((1, DIM), lambda i: (0, 0)),
        ],
        out_specs=pl.BlockSpec((blk, DIM), lambda i: (i, 0)),
        out_shape=jax.ShapeDtypeStruct((rows, DIM), jnp.float32),
    )(psum, idx, gamma.reshape(1, DIM), beta.reshape(1, DIM))


def kernel(x_s, x_t, W, gamma, beta):
    b = x_s.shape[0]
    rows = 2 * b
    idx = jnp.concatenate([x_s, x_t], axis=0)
    idx_flat = jnp.pad(idx, ((0, 0), (0, LP - L))).reshape(-1)
    psum = _sc_pool(W, idx_flat, rows)
    out = _tc_norm(psum, idx, gamma, beta, rows)
    return out[:b], out[b:]
